# baseline (speedup 1.0000x reference)
import functools

import jax
import jax.numpy as jnp
from jax import lax
from jax.experimental import pallas as pl
from jax.experimental.pallas import tpu as pltpu
from jax.experimental.pallas import tpu_sc as plsc

B, NN, F, V, D = 16384, 13, 26, 100000, 32
NC, NS = 2, 16
NW = NC * NS
RPW = B // NW
GCH = 128
NCH = RPW // GCH
T = 1 + NN + F
H = D // 16


VB = (V // 128)
UNITS = F * VB
UPW = 636
VT = VB * 128


def _cvt_body(tabt, tailc, outc, src0, src1, cmp0, cmp1, tvm,
              si0, si1, so0, so1):
    wid = lax.axis_index("s") * NC + lax.axis_index("c")
    srcs = (src0, src1)
    cmps = (cmp0, cmp1)
    sis = (si0, si1)
    sos = (so0, so1)

    @pl.when(wid < F)
    def _tail():
        pltpu.sync_copy(tailc.at[wid], tvm)
        pltpu.sync_copy(tvm, outc.at[wid, pl.ds(VB * 32, 8)])

    def coords(u):
        uid = jnp.minimum(wid * UPW + u, UNITS - 1)
        f = uid // VB
        vb = uid - f * VB
        return f, vb

    dv = [lax.iota(jnp.int32, 16) + h * 16 for h in range(H)]

    f0, vb0 = coords(0)
    pltpu.async_copy(tabt.at[f0, :, pl.ds(vb0 * 128, 128)], src0, si0)

    def u2_body(u2, carry):
        for b in range(2):
            u = u2 * 2 + b
            pltpu.make_async_copy(
                tabt.at[0, :, pl.ds(0, 128)], srcs[b], sis[b]).wait()
            fn, vbn = coords(u + 1)

            @pl.when(u + 1 < UPW)
            def _start_next():
                pltpu.async_copy(
                    tabt.at[fn, :, pl.ds(vbn * 128, 128)],
                    srcs[1 - b], sis[1 - b])

            @pl.when(u2 > 0)
            def _drain_prev():
                pltpu.make_async_copy(
                    cmps[b], outc.at[0, pl.ds(0, 32)], sos[b]).wait()

            for vm in range(128):
                vmv = jnp.full((16,), vm, jnp.int32)
                for h in range(H):
                    vals = plsc.load_gather(srcs[b], [dv[h], vmv])
                    cmps[b][vm // 4, pl.ds((vm % 4) * 32 + h * 16, 16)] = vals

            f, vb = coords(u)
            pltpu.async_copy(cmps[b], outc.at[f, pl.ds(vb * 32, 32)], sos[b])
        return carry

    lax.fori_loop(0, UPW // 2, u2_body, 0)
    pltpu.make_async_copy(cmp0, outc.at[0, pl.ds(0, 32)], so0).wait()
    pltpu.make_async_copy(cmp1, outc.at[0, pl.ds(0, 32)], so1).wait()


@functools.cache
def _cvt_call():
    mesh = plsc.VectorSubcoreMesh(core_axis_name="c", subcore_axis_name="s")
    return pl.kernel(
        _cvt_body,
        mesh=mesh,
        compiler_params=pltpu.CompilerParams(use_tc_tiling_on_sc=True,
                                             needs_layout_passes=False),
        out_type=jax.ShapeDtypeStruct((F, V // 4, 128), jnp.float32),
        scratch_types=[
            pltpu.VMEM((D, 128), jnp.float32),
            pltpu.VMEM((D, 128), jnp.float32),
            pltpu.VMEM((32, 128), jnp.float32),
            pltpu.VMEM((32, 128), jnp.float32),
            pltpu.VMEM((8, 128), jnp.float32),
            pltpu.SemaphoreType.DMA,
            pltpu.SemaphoreType.DMA,
            pltpu.SemaphoreType.DMA,
            pltpu.SemaphoreType.DMA,
        ],
    )


def _sc_body(tab, xcat, xnum, wts, bias, cls, out,
             idxb, xcb, rows, numbuf, xnb, wbuf, bbuf, clsb, sem):
    wid = lax.axis_index("s") * NC + lax.axis_index("c")
    base = wid * RPW

    pltpu.sync_copy(xcat.at[pl.ds(base, RPW)], xcb)
    basev = lax.iota(jnp.int32, 16)

    def f_body(f, carry):
        fv16 = jnp.full((16,), f, jnp.int32)
        for j in range(RPW // 16):
            iv = basev + (j * 16)
            idxb[pl.ds(j * 16, 16)] = plsc.load_gather(xcb, [iv, fv16])
        pltpu.async_copy(tab.at[f].at[idxb], rows, sem).wait()
        pltpu.sync_copy(rows, out.at[pl.ds(base, RPW), 1 + NN + f])
        return carry

    lax.fori_loop(0, F, f_body, 0)

    pltpu.sync_copy(wts, wbuf)
    pltpu.sync_copy(bias, bbuf)
    pltpu.sync_copy(cls, clsb)

    for c in range(NCH):
        pltpu.sync_copy(xnum.at[pl.ds(base + c * GCH, GCH)], xnb)

        cv = [clsb[0, pl.ds(h * 16, 16)] for h in range(H)]

        def cls_iter(i, carry, cv=cv):
            for h in range(H):
                numbuf[i, 0, pl.ds(h * 16, 16)] = cv[h]
            return carry

        lax.fori_loop(0, GCH, cls_iter, 0)

        for n in range(NN):
            wv = [wbuf[n, pl.ds(h * 16, 16)] for h in range(H)]
            bv = [bbuf[n, pl.ds(h * 16, 16)] for h in range(H)]
            nv = jnp.full((16,), n, jnp.int32)

            def num_iter(i, carry, wv=wv, bv=bv, nv=nv, n=n):
                iv = jnp.full((16,), i, jnp.int32)
                sv = plsc.load_gather(xnb, [iv, nv])
                for h in range(H):
                    numbuf[i, 1 + n, pl.ds(h * 16, 16)] = sv * wv[h] + bv[h]
                return carry

            lax.fori_loop(0, GCH, num_iter, 0)

        pltpu.sync_copy(numbuf, out.at[pl.ds(base + c * GCH, GCH), pl.ds(0, 1 + NN)])


@functools.cache
def _sc_call():
    mesh = plsc.VectorSubcoreMesh(core_axis_name="c", subcore_axis_name="s")
    return pl.kernel(
        _sc_body,
        mesh=mesh,
        compiler_params=pltpu.CompilerParams(use_tc_tiling_on_sc=False,
                                             needs_layout_passes=False),
        out_type=jax.ShapeDtypeStruct((B, T, D), jnp.float32),
        scratch_types=[
            pltpu.VMEM((RPW,), jnp.int32),
            pltpu.VMEM((RPW, F), jnp.int32),
            pltpu.VMEM((RPW, D), jnp.float32),
            pltpu.VMEM((GCH, 1 + NN, D), jnp.float32),
            pltpu.VMEM((GCH, NN), jnp.float32),
            pltpu.VMEM((NN, D), jnp.float32),
            pltpu.VMEM((NN, D), jnp.float32),
            pltpu.VMEM((1, D), jnp.float32),
            pltpu.SemaphoreType.DMA,
        ],
    )


@jax.jit
def _impl(x_num, x_cat, num_weights, num_bias, cat_tables, cls_token):
    cls = cls_token.reshape(1, D)
    tabt = jnp.transpose(cat_tables, (0, 2, 1))
    tailc = cat_tables[:, VT:, :].reshape(F, 8, 128)
    outc = _cvt_call()(tabt, tailc)
    tab = outc.reshape(F, V, D)
    return _sc_call()(tab, x_cat, x_num, num_weights, num_bias, cls)


def kernel(x_num, x_cat, num_weights, num_bias, cat_tables, cls_token):
    return _impl(x_num, x_cat, num_weights, num_bias, cat_tables, cls_token)


# speedup vs baseline: 1.9390x; 1.9390x over previous
import functools

import jax
import jax.numpy as jnp
from jax import lax
from jax.experimental import pallas as pl
from jax.experimental.pallas import tpu as pltpu
from jax.experimental.pallas import tpu_sc as plsc

B, NN, F, V, D = 16384, 13, 26, 100000, 32
NC, NS = 2, 16
NW = NC * NS
RPW = B // NW
GCH = 128
NCH = RPW // GCH
T = 1 + NN + F
H = D // 16


def _sc_body(tab, xcat, xnum, wts, bias, cls, out,
             idxb, xcb, rows, numbuf, xnb, wbuf, bbuf, clsb, sem):
    wid = lax.axis_index("s") * NC + lax.axis_index("c")
    base = wid * RPW

    pltpu.sync_copy(xcat.at[pl.ds(base, RPW)], xcb)
    basev = lax.iota(jnp.int32, 16)

    def f_body(f, carry):
        fv16 = jnp.full((16,), f, jnp.int32)
        for j in range(RPW // 16):
            iv = basev + (j * 16)
            idxb[pl.ds(j * 16, 16)] = plsc.load_gather(xcb, [iv, fv16]) * 4
        pltpu.async_copy(tab.at[f].at[idxb], rows, sem).wait()
        pltpu.sync_copy(rows, out.at[pl.ds(base, RPW), 1 + NN + f, pl.ds(0, D)])
        return carry

    lax.fori_loop(0, F, f_body, 0)

    pltpu.sync_copy(wts, wbuf)
    pltpu.sync_copy(bias, bbuf)
    pltpu.sync_copy(cls, clsb)

    for c in range(NCH):
        pltpu.sync_copy(xnum.at[pl.ds(base + c * GCH, GCH)], xnb)

        cv = [clsb[0, pl.ds(h * 16, 16)] for h in range(H)]

        def cls_iter(i, carry, cv=cv):
            for h in range(H):
                numbuf[i, 0, pl.ds(h * 16, 16)] = cv[h]
            return carry

        lax.fori_loop(0, GCH, cls_iter, 0)

        for n in range(NN):
            wv = [wbuf[n, pl.ds(h * 16, 16)] for h in range(H)]
            bv = [bbuf[n, pl.ds(h * 16, 16)] for h in range(H)]
            nv = jnp.full((16,), n, jnp.int32)

            def num_iter(i, carry, wv=wv, bv=bv, nv=nv, n=n):
                iv = jnp.full((16,), i, jnp.int32)
                sv = plsc.load_gather(xnb, [iv, nv])
                for h in range(H):
                    numbuf[i, 1 + n, pl.ds(h * 16, 16)] = sv * wv[h] + bv[h]
                return carry

            lax.fori_loop(0, GCH, num_iter, 0)

        pltpu.sync_copy(numbuf, out.at[pl.ds(base + c * GCH, GCH), pl.ds(0, 1 + NN),
                                       pl.ds(0, D)])


@functools.cache
def _sc_call():
    mesh = plsc.VectorSubcoreMesh(core_axis_name="c", subcore_axis_name="s")
    return pl.kernel(
        _sc_body,
        mesh=mesh,
        compiler_params=pltpu.CompilerParams(use_tc_tiling_on_sc=False,
                                             needs_layout_passes=False),
        out_type=jax.ShapeDtypeStruct((B, T, 128), jnp.float32),
        scratch_types=[
            pltpu.VMEM((RPW,), jnp.int32),
            pltpu.VMEM((RPW, F), jnp.int32),
            pltpu.VMEM((RPW, D), jnp.float32),
            pltpu.VMEM((GCH, 1 + NN, D), jnp.float32),
            pltpu.VMEM((GCH, NN), jnp.float32),
            pltpu.VMEM((NN, D), jnp.float32),
            pltpu.VMEM((NN, D), jnp.float32),
            pltpu.VMEM((1, D), jnp.float32),
            pltpu.SemaphoreType.DMA,
        ],
    )


@jax.jit
def _impl(x_num, x_cat, num_weights, num_bias, cat_tables, cls_token):
    cls = cls_token.reshape(1, D)
    tabp = jnp.pad(cat_tables, ((0, 0), (0, 0), (0, 128 - D))).reshape(F, V * 4, D)
    outp = _sc_call()(tabp, x_cat, x_num, num_weights, num_bias, cls)
    return outp[:, :, :D]


def kernel(x_num, x_cat, num_weights, num_bias, cat_tables, cls_token):
    return _impl(x_num, x_cat, num_weights, num_bias, cat_tables, cls_token)
